# initial kernel scaffold (unmeasured)
import jax
import jax.numpy as jnp
from jax import lax
from jax.experimental import pallas as pl
from jax.experimental.pallas import tpu as pltpu

N_DEV = 16
SQ = 512
D = 1024
NH = 8
DH = 128
SCALE = 0.08838834764831843


def _contrib(x_t, wq, wk, wv, wo):
    q = jnp.dot(x_t, wq, preferred_element_type=jnp.float32).astype(jnp.bfloat16)
    k = jnp.dot(x_t, wk, preferred_element_type=jnp.float32).astype(jnp.bfloat16)
    v = jnp.dot(x_t, wv, preferred_element_type=jnp.float32).astype(jnp.bfloat16)
    acc = None
    for h in range(NH):
        sl = slice(h * DH, (h + 1) * DH)
        qh, kh, vh = q[:, sl], k[:, sl], v[:, sl]
        s = lax.dot_general(
            qh, kh, (((1,), (1,)), ((), ())),
            preferred_element_type=jnp.float32,
        ) * SCALE
        m = jnp.max(s, axis=-1, keepdims=True)
        p = jnp.exp(s - m)
        l = jnp.sum(p, axis=-1, keepdims=True)
        o = jnp.dot(p.astype(jnp.bfloat16), vh, preferred_element_type=jnp.float32)
        o = (o / l).astype(jnp.bfloat16)
        c = jnp.dot(o, wo[sl, :], preferred_element_type=jnp.float32)
        acc = c if acc is None else acc + c
    return acc


def kernel(x, Wq, Wo, Wk, Wv):
    xb = x.reshape(SQ, D).astype(jnp.bfloat16)
    wqb = Wq.astype(jnp.bfloat16)
    wkb = Wk.astype(jnp.bfloat16)
    wvb = Wv.astype(jnp.bfloat16)
    wob = Wo.astype(jnp.bfloat16)

    def body(x_ref, wq_ref, wk_ref, wv_ref, wo_ref, out_ref,
             xbuf, psend, pbuf, xs_sem, xr_sem, ps_sem, pr_sem):
        my = lax.axis_index("i")
        left = lax.rem(my + N_DEV - 1, N_DEV)
        right = lax.rem(my + 1, N_DEV)

        barrier = pltpu.get_barrier_semaphore()
        pl.semaphore_signal(barrier, inc=1, device_id=(left,),
                            device_id_type=pl.DeviceIdType.MESH)
        pl.semaphore_signal(barrier, inc=1, device_id=(right,),
                            device_id_type=pl.DeviceIdType.MESH)
        pl.semaphore_wait(barrier, 2)

        xbuf[0] = x_ref[...]
        pbuf[1] = jnp.zeros((SQ, D), jnp.float32)

        def step(s, carry):
            slot = lax.rem(s, 2)
            nslot = lax.rem(s + 1, 2)
            x_rdma = pltpu.make_async_remote_copy(
                src_ref=xbuf.at[slot],
                dst_ref=xbuf.at[nslot],
                send_sem=xs_sem.at[slot],
                recv_sem=xr_sem.at[nslot],
                device_id=(right,),
                device_id_type=pl.DeviceIdType.MESH,
            )
            x_rdma.start()
            x_rdma.wait()
            x_t = pl.load(xbuf, (pl.dslice(nslot, 1), slice(None), slice(None)))[0]
            c = _contrib(x_t, wq_ref[...], wk_ref[...], wv_ref[...], wo_ref[...])
            prev = pl.load(pbuf, (pl.dslice(nslot, 1), slice(None), slice(None)))[0]
            c = c + prev
            pl.store(psend, (pl.dslice(slot, 1), slice(None), slice(None)), c[None])
            p_rdma = pltpu.make_async_remote_copy(
                src_ref=psend.at[slot],
                dst_ref=pbuf.at[slot],
                send_sem=ps_sem.at[slot],
                recv_sem=pr_sem.at[slot],
                device_id=(right,),
                device_id_type=pl.DeviceIdType.MESH,
            )
            p_rdma.start()
            p_rdma.wait()
            return carry

        lax.fori_loop(0, N_DEV - 1, step, 0)

        own = _contrib(x_ref[...], wq_ref[...], wk_ref[...], wv_ref[...],
                       wo_ref[...])
        out_ref[...] = own + pbuf[0]

    out = pl.pallas_call(
        body,
        out_shape=jax.ShapeDtypeStruct((SQ, D), jnp.float32),
        in_specs=[pl.BlockSpec(memory_space=pltpu.VMEM)] * 5,
        out_specs=pl.BlockSpec(memory_space=pltpu.VMEM),
        scratch_shapes=[
            pltpu.VMEM((2, SQ, D), jnp.bfloat16),
            pltpu.VMEM((2, SQ, D), jnp.float32),
            pltpu.VMEM((2, SQ, D), jnp.float32),
            pltpu.SemaphoreType.DMA((2,)),
            pltpu.SemaphoreType.DMA((2,)),
            pltpu.SemaphoreType.DMA((2,)),
            pltpu.SemaphoreType.DMA((2,)),
        ],
        compiler_params=pltpu.CompilerParams(collective_id=0),
    )(xb, wqb, wkb, wvb, wob)
    return out.reshape(1, SQ, D)


# baseline (device time: 746261 ns/iter reference)
import jax
import jax.numpy as jnp
from jax import lax
from jax.experimental import pallas as pl
from jax.experimental.pallas import tpu as pltpu

N_DEV = 16
SQ = 512
D = 1024
NH = 8
DH = 128
SCALE = 0.08838834764831843


def _contrib(x_t, wq, wk, wv, wo):
    q = jnp.dot(x_t, wq, preferred_element_type=jnp.float32).astype(jnp.bfloat16)
    k = jnp.dot(x_t, wk, preferred_element_type=jnp.float32).astype(jnp.bfloat16)
    v = jnp.dot(x_t, wv, preferred_element_type=jnp.float32).astype(jnp.bfloat16)
    acc = None
    for h in range(NH):
        sl = slice(h * DH, (h + 1) * DH)
        qh, kh, vh = q[:, sl], k[:, sl], v[:, sl]
        s = lax.dot_general(
            qh, kh, (((1,), (1,)), ((), ())),
            preferred_element_type=jnp.float32,
        ) * SCALE
        m = jnp.max(s, axis=-1, keepdims=True)
        p = jnp.exp(s - m)
        l = jnp.sum(p, axis=-1, keepdims=True)
        o = jnp.dot(p.astype(jnp.bfloat16), vh, preferred_element_type=jnp.float32)
        o = (o / l).astype(jnp.bfloat16)
        c = jnp.dot(o, wo[sl, :], preferred_element_type=jnp.float32)
        acc = c if acc is None else acc + c
    return acc


def kernel(x, Wq, Wo, Wk, Wv):
    xb = x.reshape(SQ, D).astype(jnp.bfloat16)
    wqb = Wq.astype(jnp.bfloat16)
    wkb = Wk.astype(jnp.bfloat16)
    wvb = Wv.astype(jnp.bfloat16)
    wob = Wo.astype(jnp.bfloat16)

    def body(x_ref, wq_ref, wk_ref, wv_ref, wo_ref, out_ref,
             xbuf, psend, pbuf, xs_sem, xr_sem, ps_sem, pr_sem):
        my = lax.axis_index("i")
        left = lax.rem(my + N_DEV - 1, N_DEV)
        right = lax.rem(my + 1, N_DEV)

        barrier = pltpu.get_barrier_semaphore()
        pl.semaphore_signal(barrier, inc=1, device_id=(left,),
                            device_id_type=pl.DeviceIdType.MESH)
        pl.semaphore_signal(barrier, inc=1, device_id=(right,),
                            device_id_type=pl.DeviceIdType.MESH)
        pl.semaphore_wait(barrier, 2)

        xbuf[0] = x_ref[...]
        pbuf[1] = jnp.zeros((SQ, D), jnp.float32)

        def step(s, carry):
            slot = lax.rem(s, 2)
            nslot = lax.rem(s + 1, 2)
            x_rdma = pltpu.make_async_remote_copy(
                src_ref=xbuf.at[slot],
                dst_ref=xbuf.at[nslot],
                send_sem=xs_sem.at[slot],
                recv_sem=xr_sem.at[nslot],
                device_id=(right,),
                device_id_type=pl.DeviceIdType.MESH,
            )
            x_rdma.start()
            x_rdma.wait()
            x_t = xbuf[nslot]
            c = _contrib(x_t, wq_ref[...], wk_ref[...], wv_ref[...], wo_ref[...])
            c = c + pbuf[nslot]
            psend[slot] = c
            p_rdma = pltpu.make_async_remote_copy(
                src_ref=psend.at[slot],
                dst_ref=pbuf.at[slot],
                send_sem=ps_sem.at[slot],
                recv_sem=pr_sem.at[slot],
                device_id=(right,),
                device_id_type=pl.DeviceIdType.MESH,
            )
            p_rdma.start()
            p_rdma.wait()
            return carry

        lax.fori_loop(0, N_DEV - 1, step, 0)

        own = _contrib(x_ref[...], wq_ref[...], wk_ref[...], wv_ref[...],
                       wo_ref[...])
        out_ref[...] = own + pbuf[0]

    out = pl.pallas_call(
        body,
        out_shape=jax.ShapeDtypeStruct((SQ, D), jnp.float32),
        in_specs=[pl.BlockSpec(memory_space=pltpu.VMEM)] * 5,
        out_specs=pl.BlockSpec(memory_space=pltpu.VMEM),
        scratch_shapes=[
            pltpu.VMEM((2, SQ, D), jnp.bfloat16),
            pltpu.VMEM((2, SQ, D), jnp.float32),
            pltpu.VMEM((2, SQ, D), jnp.float32),
            pltpu.SemaphoreType.DMA((2,)),
            pltpu.SemaphoreType.DMA((2,)),
            pltpu.SemaphoreType.DMA((2,)),
            pltpu.SemaphoreType.DMA((2,)),
        ],
        compiler_params=pltpu.CompilerParams(collective_id=0),
    )(xb, wqb, wkb, wvb, wob)
    return out.reshape(1, SQ, D)


# device time: 382708 ns/iter; 1.9499x vs baseline; 1.9499x over previous
import jax
import jax.numpy as jnp
from jax import lax
from jax.experimental import pallas as pl
from jax.experimental.pallas import tpu as pltpu

N_DEV = 16
SQ = 512
D = 1024
NH = 8
DH = 128
SCALE = 0.08838834764831843


def _contrib(x_t, wq, wk, wv, wo):
    q = jnp.dot(x_t, wq, preferred_element_type=jnp.float32).astype(jnp.bfloat16)
    k = jnp.dot(x_t, wk, preferred_element_type=jnp.float32).astype(jnp.bfloat16)
    v = jnp.dot(x_t, wv, preferred_element_type=jnp.float32).astype(jnp.bfloat16)
    acc = None
    for h in range(NH):
        sl = slice(h * DH, (h + 1) * DH)
        qh, kh, vh = q[:, sl], k[:, sl], v[:, sl]
        s = lax.dot_general(
            qh, kh, (((1,), (1,)), ((), ())),
            preferred_element_type=jnp.float32,
        ) * SCALE
        m = jnp.max(s, axis=-1, keepdims=True)
        p = jnp.exp(s - m)
        l = jnp.sum(p, axis=-1, keepdims=True)
        o = jnp.dot(p.astype(jnp.bfloat16), vh, preferred_element_type=jnp.float32)
        o = (o / l).astype(jnp.bfloat16)
        c = jnp.dot(o, wo[sl, :], preferred_element_type=jnp.float32)
        acc = c if acc is None else acc + c
    return acc


def kernel(x, Wq, Wo, Wk, Wv):
    xb = x.reshape(SQ, D).astype(jnp.bfloat16)
    wqb = Wq.astype(jnp.bfloat16)
    wkb = Wk.astype(jnp.bfloat16)
    wvb = Wv.astype(jnp.bfloat16)
    wob = Wo.astype(jnp.bfloat16)

    def body(x_ref, wq_ref, wk_ref, wv_ref, wo_ref, out_ref,
             xgat, psend, precv, xs_sem, xr_sem, ps_sem, pr_sem):
        my = lax.axis_index("i")
        left = lax.rem(my + N_DEV - 1, N_DEV)
        right = lax.rem(my + 1, N_DEV)

        def x_hop(s):
            return pltpu.make_async_remote_copy(
                src_ref=xgat.at[s],
                dst_ref=xgat.at[s + 1],
                send_sem=xs_sem.at[s],
                recv_sem=xr_sem.at[s],
                device_id=(right,),
                device_id_type=pl.DeviceIdType.MESH,
            )

        def p_hop(s):
            return pltpu.make_async_remote_copy(
                src_ref=psend.at[lax.rem(s, 4)],
                dst_ref=precv.at[s + 1],
                send_sem=ps_sem.at[s],
                recv_sem=pr_sem.at[s],
                device_id=(right,),
                device_id_type=pl.DeviceIdType.MESH,
            )

        barrier = pltpu.get_barrier_semaphore()
        pl.semaphore_signal(barrier, inc=1, device_id=(left,),
                            device_id_type=pl.DeviceIdType.MESH)
        pl.semaphore_signal(barrier, inc=1, device_id=(right,),
                            device_id_type=pl.DeviceIdType.MESH)
        pl.semaphore_wait(barrier, 2)

        xgat[0] = x_ref[...]
        precv[0] = jnp.zeros((SQ, D), jnp.bfloat16)
        x_hop(0).start()

        def step(s, carry):
            x_hop(s).wait_recv()

            @pl.when(s < N_DEV - 2)
            def _():
                x_hop(s + 1).start()

            c = _contrib(xgat[s + 1], wq_ref[...], wk_ref[...],
                         wv_ref[...], wo_ref[...])

            @pl.when(s > 0)
            def _():
                p_hop(s - 1).wait_recv()

            c = c + precv[s].astype(jnp.float32)

            @pl.when(s >= 4)
            def _():
                p_hop(s - 4).wait_send()

            psend[lax.rem(s, 4)] = c.astype(jnp.bfloat16)
            p_hop(s).start()
            return carry

        lax.fori_loop(0, N_DEV - 1, step, 0)

        p_hop(N_DEV - 2).wait_recv()
        own = _contrib(x_ref[...], wq_ref[...], wk_ref[...], wv_ref[...],
                       wo_ref[...])
        out_ref[...] = own + precv[N_DEV - 1].astype(jnp.float32)

        def drain_x(s, carry):
            x_hop(s).wait_send()
            return carry

        lax.fori_loop(0, N_DEV - 1, drain_x, 0)

        def drain_p(s, carry):
            p_hop(s).wait_send()
            return carry

        lax.fori_loop(N_DEV - 5, N_DEV - 1, drain_p, 0)

    out = pl.pallas_call(
        body,
        out_shape=jax.ShapeDtypeStruct((SQ, D), jnp.float32),
        in_specs=[pl.BlockSpec(memory_space=pltpu.VMEM)] * 5,
        out_specs=pl.BlockSpec(memory_space=pltpu.VMEM),
        scratch_shapes=[
            pltpu.VMEM((N_DEV, SQ, D), jnp.bfloat16),
            pltpu.VMEM((4, SQ, D), jnp.bfloat16),
            pltpu.VMEM((N_DEV, SQ, D), jnp.bfloat16),
            pltpu.SemaphoreType.DMA((N_DEV - 1,)),
            pltpu.SemaphoreType.DMA((N_DEV - 1,)),
            pltpu.SemaphoreType.DMA((N_DEV - 1,)),
            pltpu.SemaphoreType.DMA((N_DEV - 1,)),
        ],
        compiler_params=pltpu.CompilerParams(
            collective_id=0,
            vmem_limit_bytes=100 * 1024 * 1024,
        ),
    )(xb, wqb, wkb, wvb, wob)
    return out.reshape(1, SQ, D)


# device time: 307867 ns/iter; 2.4240x vs baseline; 1.2431x over previous
import jax
import jax.numpy as jnp
from jax import lax
from jax.experimental import pallas as pl
from jax.experimental.pallas import tpu as pltpu

N_DEV = 16
SQ = 512
D = 1024
NH = 8
DH = 128
SCALE = 0.08838834764831843


def _contrib(x_t, wq, wk, wv, wo):
    q = jnp.dot(x_t, wq, preferred_element_type=jnp.float32).astype(jnp.bfloat16)
    k = jnp.dot(x_t, wk, preferred_element_type=jnp.float32).astype(jnp.bfloat16)
    v = jnp.dot(x_t, wv, preferred_element_type=jnp.float32).astype(jnp.bfloat16)
    acc = None
    for h in range(NH):
        sl = slice(h * DH, (h + 1) * DH)
        qh, kh, vh = q[:, sl], k[:, sl], v[:, sl]
        s = lax.dot_general(
            qh, kh, (((1,), (1,)), ((), ())),
            preferred_element_type=jnp.float32,
        ) * SCALE
        m = jnp.max(s, axis=-1, keepdims=True)
        p = jnp.exp(s - m)
        l = jnp.sum(p, axis=-1, keepdims=True)
        o = jnp.dot(p.astype(jnp.bfloat16), vh, preferred_element_type=jnp.float32)
        o = (o / l).astype(jnp.bfloat16)
        c = jnp.dot(o, wo[sl, :], preferred_element_type=jnp.float32)
        acc = c if acc is None else acc + c
    return acc


def kernel(x, Wq, Wo, Wk, Wv):
    xb = x.reshape(SQ, D).astype(jnp.bfloat16)
    wqb = Wq.astype(jnp.bfloat16)
    wkb = Wk.astype(jnp.bfloat16)
    wvb = Wv.astype(jnp.bfloat16)
    wob = Wo.astype(jnp.bfloat16)

    def body(x_ref, wq_ref, wk_ref, wv_ref, wo_ref, out_ref,
             xgat, psend, precv, xs_sem, xr_sem, xsl_sem, xrl_sem,
             ps_sem, pr_sem):
        my = lax.axis_index("i")
        left = lax.rem(my + N_DEV - 1, N_DEV)
        right = lax.rem(my + 1, N_DEV)

        def x_r_hop(r):
            return pltpu.make_async_remote_copy(
                src_ref=xgat.at[r],
                dst_ref=xgat.at[r + 1],
                send_sem=xs_sem.at[r],
                recv_sem=xr_sem.at[r],
                device_id=(right,),
                device_id_type=pl.DeviceIdType.MESH,
            )

        def x_l_hop(l):
            return pltpu.make_async_remote_copy(
                src_ref=xgat.at[lax.rem(N_DEV - l, N_DEV)],
                dst_ref=xgat.at[N_DEV - 1 - l],
                send_sem=xsl_sem.at[l],
                recv_sem=xrl_sem.at[l],
                device_id=(left,),
                device_id_type=pl.DeviceIdType.MESH,
            )

        def p_hop(s):
            return pltpu.make_async_remote_copy(
                src_ref=psend.at[lax.rem(s, 4)],
                dst_ref=precv.at[s + 1],
                send_sem=ps_sem.at[s],
                recv_sem=pr_sem.at[s],
                device_id=(right,),
                device_id_type=pl.DeviceIdType.MESH,
            )

        barrier = pltpu.get_barrier_semaphore()
        pl.semaphore_signal(barrier, inc=1, device_id=(left,),
                            device_id_type=pl.DeviceIdType.MESH)
        pl.semaphore_signal(barrier, inc=1, device_id=(right,),
                            device_id_type=pl.DeviceIdType.MESH)
        pl.semaphore_wait(barrier, 2)

        xgat[0] = x_ref[...]
        precv[0] = jnp.zeros((SQ, D), jnp.bfloat16)
        x_r_hop(0).start()
        x_l_hop(0).start()

        def step(s, carry):
            @pl.when(s < 8)
            def _():
                x_r_hop(lax.min(s, 7)).wait_recv()

            @pl.when(s < 7)
            def _():
                x_r_hop(lax.min(s + 1, 7)).start()
                x_l_hop(lax.min(s, 6)).wait_recv()

            @pl.when(s < 6)
            def _():
                x_l_hop(lax.min(s + 1, 6)).start()

            c = _contrib(xgat[s + 1], wq_ref[...], wk_ref[...],
                         wv_ref[...], wo_ref[...])

            @pl.when(s > 0)
            def _():
                p_hop(s - 1).wait_recv()

            c = c + precv[s].astype(jnp.float32)

            @pl.when(s >= 4)
            def _():
                p_hop(s - 4).wait_send()

            psend[lax.rem(s, 4)] = c.astype(jnp.bfloat16)
            p_hop(s).start()
            return carry

        lax.fori_loop(0, N_DEV - 1, step, 0)

        own = _contrib(x_ref[...], wq_ref[...], wk_ref[...], wv_ref[...],
                       wo_ref[...])
        p_hop(N_DEV - 2).wait_recv()
        out_ref[...] = own + precv[N_DEV - 1].astype(jnp.float32)

        def drain_xr(s, carry):
            x_r_hop(s).wait_send()
            return carry

        lax.fori_loop(0, 8, drain_xr, 0)

        def drain_xl(s, carry):
            x_l_hop(s).wait_send()
            return carry

        lax.fori_loop(0, 7, drain_xl, 0)

        def drain_p(s, carry):
            p_hop(s).wait_send()
            return carry

        lax.fori_loop(N_DEV - 5, N_DEV - 1, drain_p, 0)

    out = pl.pallas_call(
        body,
        out_shape=jax.ShapeDtypeStruct((SQ, D), jnp.float32),
        in_specs=[pl.BlockSpec(memory_space=pltpu.VMEM)] * 5,
        out_specs=pl.BlockSpec(memory_space=pltpu.VMEM),
        scratch_shapes=[
            pltpu.VMEM((N_DEV, SQ, D), jnp.bfloat16),
            pltpu.VMEM((4, SQ, D), jnp.bfloat16),
            pltpu.VMEM((N_DEV, SQ, D), jnp.bfloat16),
            pltpu.SemaphoreType.DMA((8,)),
            pltpu.SemaphoreType.DMA((8,)),
            pltpu.SemaphoreType.DMA((7,)),
            pltpu.SemaphoreType.DMA((7,)),
            pltpu.SemaphoreType.DMA((N_DEV - 1,)),
            pltpu.SemaphoreType.DMA((N_DEV - 1,)),
        ],
        compiler_params=pltpu.CompilerParams(
            collective_id=0,
            vmem_limit_bytes=100 * 1024 * 1024,
        ),
    )(xb, wqb, wkb, wvb, wob)
    return out.reshape(1, SQ, D)


# device time: 306232 ns/iter; 2.4369x vs baseline; 1.0053x over previous
import jax
import jax.numpy as jnp
from jax import lax
from jax.experimental import pallas as pl
from jax.experimental.pallas import tpu as pltpu

N_DEV = 16
SQ = 512
D = 1024
NH = 8
DH = 128
SCALE = 0.08838834764831843


def _contrib(x_t, wqkv, wo):
    qkv = jnp.dot(x_t, wqkv, preferred_element_type=jnp.float32).astype(
        jnp.bfloat16)
    acc = None
    for h in range(NH):
        qh = qkv[:, h * DH:(h + 1) * DH]
        kh = qkv[:, D + h * DH:D + (h + 1) * DH]
        vh = qkv[:, 2 * D + h * DH:2 * D + (h + 1) * DH]
        s = lax.dot_general(
            qh, kh, (((1,), (1,)), ((), ())),
            preferred_element_type=jnp.float32,
        ) * SCALE
        p = jnp.exp(s)
        l = jnp.sum(p, axis=-1, keepdims=True)
        o = jnp.dot(p.astype(jnp.bfloat16), vh, preferred_element_type=jnp.float32)
        o = (o / l).astype(jnp.bfloat16)
        c = jnp.dot(o, wo[h * DH:(h + 1) * DH, :],
                    preferred_element_type=jnp.float32)
        acc = c if acc is None else acc + c
    return acc


def kernel(x, Wq, Wo, Wk, Wv):
    xb = x.reshape(SQ, D).astype(jnp.bfloat16)
    wqkvb = jnp.concatenate(
        [Wq.astype(jnp.bfloat16), Wk.astype(jnp.bfloat16),
         Wv.astype(jnp.bfloat16)], axis=1)
    wob = Wo.astype(jnp.bfloat16)

    def body(x_ref, wqkv_ref, wo_ref, out_ref,
             xgat, psend, precv, xs_sem, xr_sem, xsl_sem, xrl_sem,
             ps_sem, pr_sem):
        my = lax.axis_index("i")
        left = lax.rem(my + N_DEV - 1, N_DEV)
        right = lax.rem(my + 1, N_DEV)

        def x_r_hop(r):
            return pltpu.make_async_remote_copy(
                src_ref=xgat.at[r],
                dst_ref=xgat.at[r + 1],
                send_sem=xs_sem.at[r],
                recv_sem=xr_sem.at[r],
                device_id=(right,),
                device_id_type=pl.DeviceIdType.MESH,
            )

        def x_l_hop(l):
            return pltpu.make_async_remote_copy(
                src_ref=xgat.at[lax.rem(N_DEV - l, N_DEV)],
                dst_ref=xgat.at[N_DEV - 1 - l],
                send_sem=xsl_sem.at[l],
                recv_sem=xrl_sem.at[l],
                device_id=(left,),
                device_id_type=pl.DeviceIdType.MESH,
            )

        def p_hop(s):
            return pltpu.make_async_remote_copy(
                src_ref=psend.at[lax.rem(s, 4)],
                dst_ref=precv.at[s + 1],
                send_sem=ps_sem.at[s],
                recv_sem=pr_sem.at[s],
                device_id=(right,),
                device_id_type=pl.DeviceIdType.MESH,
            )

        barrier = pltpu.get_barrier_semaphore()
        pl.semaphore_signal(barrier, inc=1, device_id=(left,),
                            device_id_type=pl.DeviceIdType.MESH)
        pl.semaphore_signal(barrier, inc=1, device_id=(right,),
                            device_id_type=pl.DeviceIdType.MESH)
        pl.semaphore_wait(barrier, 2)

        xgat[0] = x_ref[...]
        precv[0] = jnp.zeros((SQ, D), jnp.bfloat16)
        x_r_hop(0).start()
        x_l_hop(0).start()

        def step(s, carry):
            @pl.when(s < 8)
            def _():
                x_r_hop(lax.min(s, 7)).wait_recv()

            @pl.when(s < 7)
            def _():
                x_r_hop(lax.min(s + 1, 7)).start()
                x_l_hop(lax.min(s, 6)).wait_recv()

            @pl.when(s < 6)
            def _():
                x_l_hop(lax.min(s + 1, 6)).start()

            c = _contrib(xgat[s + 1], wqkv_ref[...], wo_ref[...])

            @pl.when(s > 0)
            def _():
                p_hop(s - 1).wait_recv()

            c = c + precv[s].astype(jnp.float32)

            @pl.when(s >= 4)
            def _():
                p_hop(s - 4).wait_send()

            psend[lax.rem(s, 4)] = c.astype(jnp.bfloat16)
            p_hop(s).start()
            return carry

        lax.fori_loop(0, N_DEV - 1, step, 0)

        own = _contrib(x_ref[...], wqkv_ref[...], wo_ref[...])
        p_hop(N_DEV - 2).wait_recv()
        out_ref[...] = own + precv[N_DEV - 1].astype(jnp.float32)

        def drain_xr(s, carry):
            x_r_hop(s).wait_send()
            return carry

        lax.fori_loop(0, 8, drain_xr, 0)

        def drain_xl(s, carry):
            x_l_hop(s).wait_send()
            return carry

        lax.fori_loop(0, 7, drain_xl, 0)

        def drain_p(s, carry):
            p_hop(s).wait_send()
            return carry

        lax.fori_loop(N_DEV - 5, N_DEV - 1, drain_p, 0)

    out = pl.pallas_call(
        body,
        out_shape=jax.ShapeDtypeStruct((SQ, D), jnp.float32),
        in_specs=[pl.BlockSpec(memory_space=pltpu.VMEM)] * 3,
        out_specs=pl.BlockSpec(memory_space=pltpu.VMEM),
        scratch_shapes=[
            pltpu.VMEM((N_DEV, SQ, D), jnp.bfloat16),
            pltpu.VMEM((4, SQ, D), jnp.bfloat16),
            pltpu.VMEM((N_DEV, SQ, D), jnp.bfloat16),
            pltpu.SemaphoreType.DMA((8,)),
            pltpu.SemaphoreType.DMA((8,)),
            pltpu.SemaphoreType.DMA((7,)),
            pltpu.SemaphoreType.DMA((7,)),
            pltpu.SemaphoreType.DMA((N_DEV - 1,)),
            pltpu.SemaphoreType.DMA((N_DEV - 1,)),
        ],
        compiler_params=pltpu.CompilerParams(
            collective_id=0,
            vmem_limit_bytes=100 * 1024 * 1024,
        ),
    )(xb, wqkvb, wob)
    return out.reshape(1, SQ, D)


# device time: 297507 ns/iter; 2.5084x vs baseline; 1.0293x over previous
import jax
import jax.numpy as jnp
from jax import lax
from jax.experimental import pallas as pl
from jax.experimental.pallas import tpu as pltpu

N_DEV = 16
SQ = 512
D = 1024
NH = 8
DH = 128
SCALE = 0.08838834764831843


def _contrib(x_t, wqkv, wo):
    qkv = jnp.dot(x_t, wqkv, preferred_element_type=jnp.float32).astype(
        jnp.bfloat16)
    acc = None
    for h in range(NH):
        qh = qkv[:, h * DH:(h + 1) * DH]
        kh = qkv[:, D + h * DH:D + (h + 1) * DH]
        vh = qkv[:, 2 * D + h * DH:2 * D + (h + 1) * DH]
        s = lax.dot_general(
            qh, kh, (((1,), (1,)), ((), ())),
            preferred_element_type=jnp.float32,
        ) * SCALE
        p = jnp.exp(s)
        l = jnp.sum(p, axis=-1, keepdims=True)
        o = jnp.dot(p.astype(jnp.bfloat16), vh, preferred_element_type=jnp.float32)
        o = (o / l).astype(jnp.bfloat16)
        c = jnp.dot(o, wo[h * DH:(h + 1) * DH, :],
                    preferred_element_type=jnp.float32)
        acc = c if acc is None else acc + c
    return acc


def kernel(x, Wq, Wo, Wk, Wv):
    xb = x.reshape(SQ, D).astype(jnp.bfloat16)
    wqkvb = jnp.concatenate(
        [Wq.astype(jnp.bfloat16), Wk.astype(jnp.bfloat16),
         Wv.astype(jnp.bfloat16)], axis=1)
    wob = Wo.astype(jnp.bfloat16)

    def body(x_ref, wqkv_ref, wo_ref, out_ref,
             xgat, psend, precv, xs_sem, xr_sem, xsl_sem, xrl_sem,
             ps_sem, pr_sem):
        my = lax.axis_index("i")
        left = lax.rem(my + N_DEV - 1, N_DEV)
        right = lax.rem(my + 1, N_DEV)

        def x_r_hop(r):
            return pltpu.make_async_remote_copy(
                src_ref=xgat.at[r],
                dst_ref=xgat.at[r + 1],
                send_sem=xs_sem.at[r],
                recv_sem=xr_sem.at[r],
                device_id=(right,),
                device_id_type=pl.DeviceIdType.MESH,
            )

        def x_l_hop(l):
            return pltpu.make_async_remote_copy(
                src_ref=xgat.at[lax.rem(N_DEV - l, N_DEV)],
                dst_ref=xgat.at[N_DEV - 1 - l],
                send_sem=xsl_sem.at[l],
                recv_sem=xrl_sem.at[l],
                device_id=(left,),
                device_id_type=pl.DeviceIdType.MESH,
            )

        def p_hop(s):
            return pltpu.make_async_remote_copy(
                src_ref=psend.at[lax.rem(s, 4)],
                dst_ref=precv.at[s + 1],
                send_sem=ps_sem.at[s],
                recv_sem=pr_sem.at[s],
                device_id=(right,),
                device_id_type=pl.DeviceIdType.MESH,
            )

        barrier = pltpu.get_barrier_semaphore()
        pl.semaphore_signal(barrier, inc=1, device_id=(left,),
                            device_id_type=pl.DeviceIdType.MESH)
        pl.semaphore_signal(barrier, inc=1, device_id=(right,),
                            device_id_type=pl.DeviceIdType.MESH)
        pl.semaphore_wait(barrier, 2)

        xgat[0] = x_ref[...]
        precv[0] = jnp.zeros((SQ, D), jnp.bfloat16)
        x_r_hop(0).start()
        x_l_hop(0).start()

        def step(s, carry):
            @pl.when(s < 8)
            def _():
                x_l_hop(lax.min(s, 7)).wait_recv()

            @pl.when(s < 7)
            def _():
                x_l_hop(lax.min(s + 1, 7)).start()
                x_r_hop(lax.min(s, 6)).wait_recv()

            @pl.when(s < 6)
            def _():
                x_r_hop(lax.min(s + 1, 6)).start()

            c = _contrib(xgat[s + 1], wqkv_ref[...], wo_ref[...])

            @pl.when(s > 0)
            def _():
                p_hop(s - 1).wait_recv()

            c = c + precv[s].astype(jnp.float32)

            @pl.when(s >= 4)
            def _():
                p_hop(s - 4).wait_send()

            psend[lax.rem(s, 4)] = c.astype(jnp.bfloat16)
            p_hop(s).start()
            return carry

        lax.fori_loop(0, N_DEV - 1, step, 0)

        own = _contrib(x_ref[...], wqkv_ref[...], wo_ref[...])
        p_hop(N_DEV - 2).wait_recv()
        out_ref[...] = own + precv[N_DEV - 1].astype(jnp.float32)

        def drain_xr(s, carry):
            x_r_hop(s).wait_send()
            return carry

        lax.fori_loop(0, 7, drain_xr, 0)

        def drain_xl(s, carry):
            x_l_hop(s).wait_send()
            return carry

        lax.fori_loop(0, 8, drain_xl, 0)

        def drain_p(s, carry):
            p_hop(s).wait_send()
            return carry

        lax.fori_loop(N_DEV - 5, N_DEV - 1, drain_p, 0)

    out = pl.pallas_call(
        body,
        out_shape=jax.ShapeDtypeStruct((SQ, D), jnp.float32),
        in_specs=[pl.BlockSpec(memory_space=pltpu.VMEM)] * 3,
        out_specs=pl.BlockSpec(memory_space=pltpu.VMEM),
        scratch_shapes=[
            pltpu.VMEM((N_DEV, SQ, D), jnp.bfloat16),
            pltpu.VMEM((4, SQ, D), jnp.bfloat16),
            pltpu.VMEM((N_DEV, SQ, D), jnp.bfloat16),
            pltpu.SemaphoreType.DMA((7,)),
            pltpu.SemaphoreType.DMA((7,)),
            pltpu.SemaphoreType.DMA((8,)),
            pltpu.SemaphoreType.DMA((8,)),
            pltpu.SemaphoreType.DMA((N_DEV - 1,)),
            pltpu.SemaphoreType.DMA((N_DEV - 1,)),
        ],
        compiler_params=pltpu.CompilerParams(
            collective_id=0,
            vmem_limit_bytes=100 * 1024 * 1024,
        ),
    )(xb, wqkvb, wob)
    return out.reshape(1, SQ, D)
